# Initial kernel scaffold; baseline (speedup 1.0000x reference)
import jax
import jax.numpy as jnp
from jax.experimental import pallas as pl


def kernel(x, mask, type_embedding_weight):
    raise NotImplementedError("write your pallas kernel here")


# speedup vs baseline: 5.1432x; 5.1432x over previous
import jax
import jax.numpy as jnp
from jax.experimental import pallas as pl
from jax.experimental.pallas import tpu as pltpu


def _body(mask_ref, w_ref, x_ref, o_ref):
    m0 = mask_ref[0]
    n = w_ref.shape[0]
    sel = jax.lax.broadcasted_iota(jnp.int32, (n, 1), 0) < m0
    addend = jnp.where(sel, w_ref[1:2, :], w_ref[0:1, :])
    o_ref[...] = x_ref[...] + addend[None, :, :]


def kernel(x, mask, type_embedding_weight):
    b, n, d = x.shape
    mask_i = mask.astype(jnp.int32)
    tb = 512
    grid = (b // tb,)
    return pl.pallas_call(
        _body,
        grid=grid,
        in_specs=[
            pl.BlockSpec(memory_space=pltpu.SMEM),
            pl.BlockSpec((n, d), lambda i: (0, 0)),
            pl.BlockSpec((tb, n, d), lambda i: (i, 0, 0)),
        ],
        out_specs=pl.BlockSpec((tb, n, d), lambda i: (i, 0, 0)),
        out_shape=jax.ShapeDtypeStruct((b, n, d), x.dtype),
    )(mask_i, type_embedding_weight, x)


# speedup vs baseline: 5.2537x; 1.0215x over previous
import jax
import jax.numpy as jnp
from jax.experimental import pallas as pl
from jax.experimental.pallas import tpu as pltpu


def _body(mask_ref, w_ref, x_ref, o_ref):
    m0 = mask_ref[0]
    n = w_ref.shape[0]
    sel = jax.lax.broadcasted_iota(jnp.int32, (n, 1), 0) < m0
    addend = jnp.where(sel, w_ref[1:2, :], w_ref[0:1, :])
    o_ref[...] = x_ref[...] + addend[None, :, :]


def kernel(x, mask, type_embedding_weight):
    b, n, d = x.shape
    mask_i = mask.astype(jnp.int32)
    tb = 1024
    grid = (b // tb,)
    return pl.pallas_call(
        _body,
        grid=grid,
        in_specs=[
            pl.BlockSpec(memory_space=pltpu.SMEM),
            pl.BlockSpec((n, d), lambda i: (0, 0)),
            pl.BlockSpec((tb, n, d), lambda i: (i, 0, 0)),
        ],
        out_specs=pl.BlockSpec((tb, n, d), lambda i: (i, 0, 0)),
        out_shape=jax.ShapeDtypeStruct((b, n, d), x.dtype),
    )(mask_i, type_embedding_weight, x)
